# baseline (speedup 1.0000x reference)
import functools

import jax
import jax.numpy as jnp
from jax.experimental import pallas as pl
from jax.experimental.pallas import tpu as pltpu

_EPS = 1e-5


def _stats_fold(x_ref, w_ref, b_ref, wf_ref, bf_ref, acc_ref, *, nsteps, inv_n):
    i = pl.program_id(0)

    @pl.when(i == 0)
    def _():
        acc_ref[...] = jnp.zeros_like(acc_ref)

    xb = x_ref[...]
    acc_ref[0:1, :] += jnp.sum(xb, axis=0, keepdims=True)
    acc_ref[1:2, :] += jnp.sum(xb * xb, axis=0, keepdims=True)

    @pl.when(i == nsteps - 1)
    def _():
        mean = acc_ref[0:1, :] * inv_n
        var = acc_ref[1:2, :] * inv_n - mean * mean
        rstd = jax.lax.rsqrt(var + _EPS)
        wf = w_ref[...] * rstd
        wf_ref[...] = wf
        mw = jax.lax.dot_general(mean, wf, (((1,), (1,)), ((), ())),
                                 preferred_element_type=jnp.float32)
        bf_ref[...] = b_ref[...] - mw


def _mm(x_ref, wf_ref, bf_ref, o_ref):
    o_ref[...] = jax.lax.dot_general(
        x_ref[...], wf_ref[...], (((1,), (1,)), ((), ())),
        preferred_element_type=jnp.float32) + bf_ref[...]


def kernel(nodeblocks, x, W, b):
    n, d = x.shape
    c = W.shape[0]
    blk1 = 10000
    blk2 = 10000
    nb1 = n // blk1
    nb2 = n // blk2
    b2 = b.reshape(1, c)

    wf, bf = pl.pallas_call(
        functools.partial(_stats_fold, nsteps=nb1, inv_n=1.0 / n),
        grid=(nb1,),
        in_specs=[
            pl.BlockSpec((blk1, d), lambda i: (i, 0)),
            pl.BlockSpec((c, d), lambda i: (0, 0)),
            pl.BlockSpec((1, c), lambda i: (0, 0)),
        ],
        out_specs=[
            pl.BlockSpec((c, d), lambda i: (0, 0)),
            pl.BlockSpec((1, c), lambda i: (0, 0)),
        ],
        out_shape=[
            jax.ShapeDtypeStruct((c, d), jnp.float32),
            jax.ShapeDtypeStruct((1, c), jnp.float32),
        ],
        scratch_shapes=[pltpu.VMEM((2, d), jnp.float32)],
    )(x, W, b2)

    out = pl.pallas_call(
        _mm,
        grid=(nb2,),
        in_specs=[
            pl.BlockSpec((blk2, d), lambda i: (i, 0)),
            pl.BlockSpec((c, d), lambda i: (0, 0)),
            pl.BlockSpec((1, c), lambda i: (0, 0)),
        ],
        out_specs=pl.BlockSpec((blk2, c), lambda i: (i, 0)),
        out_shape=jax.ShapeDtypeStruct((n, c), jnp.float32),
    )(x, wf, bf)
    return out
